# baseline (speedup 1.0000x reference)
import functools

import jax
import jax.numpy as jnp
from jax import lax
from jax.experimental import pallas as pl
from jax.experimental.pallas import tpu as pltpu
from jax.experimental.pallas import tpu_sc as plsc

B, J, D = 16384, 26, 128
ROW = J * D
NC, NS = 2, 16
NW = NC * NS
RPW = B // NW
CH = 8
NCHUNK = RPW // CH
GROUPS = CH * J

_mesh = plsc.VectorSubcoreMesh(core_axis_name="c", subcore_axis_name="s")


@functools.partial(
    pl.kernel,
    out_type=jax.ShapeDtypeStruct((B * ROW,), jnp.float32),
    mesh=_mesh,
    scratch_types=[
        pltpu.VMEM((4, 128), jnp.int32),
        pltpu.VMEM((4, 128), jnp.int32),
        pltpu.VMEM((512,), jnp.float32),
        pltpu.VMEM((CH * ROW,), jnp.float32),
        pltpu.SemaphoreType.DMA,
    ],
)
def _sc_scale_mul(idx_hbm, lw_hbm, rep_hbm, feat_hbm, out_hbm,
                  rep_v, eidx_v, sv_v, buf, sem):
    cid = lax.axis_index("c")
    sid = lax.axis_index("s")
    wid = sid * NC + cid

    pltpu.sync_copy(rep_hbm, rep_v)
    for q in range(4):
        pltpu.async_copy(idx_hbm.at[rep_v.at[q]], eidx_v.at[q], sem).wait()
    for q in range(4):
        pltpu.async_copy(lw_hbm.at[eidx_v.at[q]],
                         sv_v.at[pl.ds(128 * q, 128)], sem).wait()
    for t in range(512 // 16):
        wv = sv_v[pl.ds(16 * t, 16)]
        sv_v[pl.ds(16 * t, 16)] = 1.0 / (1.0 + jnp.exp(-wv))

    def chunk_body(g, carry):
        base = (wid * RPW + g * CH) * ROW
        pltpu.sync_copy(feat_hbm.at[pl.ds(base, CH * ROW)], buf)

        def grp(t, c):
            o = t * D
            j = lax.rem(t, J)
            s = sv_v[pl.ds(j * 16, 16)]
            for u in range(D // 16):
                buf[pl.ds(o + u * 16, 16)] = buf[pl.ds(o + u * 16, 16)] * s
            return c

        lax.fori_loop(0, GROUPS, grp, 0)
        pltpu.sync_copy(buf, out_hbm.at[pl.ds(base, CH * ROW)])
        return carry

    lax.fori_loop(0, NCHUNK, chunk_body, 0)


def kernel(idx, feat, layerweight):
    idxp = jnp.zeros((128,), jnp.int32).at[:J].set(
        idx.reshape(J).astype(jnp.int32))
    lwp = jnp.zeros((128,), jnp.float32).at[:100].set(layerweight)
    rep = jnp.minimum(jnp.arange(512, dtype=jnp.int32) // 16,
                      J - 1).reshape(4, 128)
    out = _sc_scale_mul(idxp, lwp, rep, feat.reshape(-1))
    return out.reshape(B, J, D)


# speedup vs baseline: 1.5057x; 1.5057x over previous
import functools

import jax
import jax.numpy as jnp
from jax import lax
from jax.experimental import pallas as pl
from jax.experimental.pallas import tpu as pltpu
from jax.experimental.pallas import tpu_sc as plsc

B, J, D = 16384, 26, 128
ROW = J * D
BB = 512

_mesh = plsc.VectorSubcoreMesh(core_axis_name="c", subcore_axis_name="s")


@functools.partial(
    pl.kernel,
    out_type=jax.ShapeDtypeStruct((ROW,), jnp.float32),
    mesh=_mesh,
    scratch_types=[
        pltpu.VMEM((J, 128), jnp.int32),
        pltpu.VMEM((J, 128), jnp.int32),
        pltpu.VMEM((ROW,), jnp.float32),
        pltpu.SemaphoreType.DMA,
    ],
)
def _sc_scales(idx_hbm, lw_hbm, rep_hbm, out_hbm, rep_v, eidx_v, sv_v, sem):
    cid = lax.axis_index("c")
    sid = lax.axis_index("s")

    @pl.when(jnp.logical_and(cid == 0, sid == 0))
    def _():
        pltpu.sync_copy(rep_hbm, rep_v)
        for q in range(J):
            pltpu.async_copy(idx_hbm.at[rep_v.at[q]], eidx_v.at[q], sem).wait()
        for q in range(J):
            pltpu.async_copy(lw_hbm.at[eidx_v.at[q]],
                             sv_v.at[pl.ds(128 * q, 128)], sem).wait()

        def sig(t, c):
            wv = sv_v[pl.ds(16 * t, 16)]
            sv_v[pl.ds(16 * t, 16)] = 1.0 / (1.0 + jnp.exp(-wv))
            return c

        lax.fori_loop(0, ROW // 16, sig, 0)
        pltpu.sync_copy(sv_v, out_hbm)


def _tc_body(scale_ref, feat_ref, out_ref):
    out_ref[...] = feat_ref[...] * scale_ref[...]


_tc_mul = pl.pallas_call(
    _tc_body,
    grid=(B // BB,),
    in_specs=[
        pl.BlockSpec((1, ROW), lambda i: (0, 0)),
        pl.BlockSpec((BB, ROW), lambda i: (i, 0)),
    ],
    out_specs=pl.BlockSpec((BB, ROW), lambda i: (i, 0)),
    out_shape=jax.ShapeDtypeStruct((B, ROW), jnp.float32),
)


def kernel(idx, feat, layerweight):
    idxp = jnp.zeros((128,), jnp.int32).at[:J].set(
        idx.reshape(J).astype(jnp.int32))
    lwp = jnp.zeros((128,), jnp.float32).at[:100].set(layerweight)
    rep = jnp.broadcast_to(jnp.arange(J, dtype=jnp.int32)[:, None], (J, 128))
    scale = _sc_scales(idxp, lwp, rep)
    out = _tc_mul(scale.reshape(1, ROW), feat.reshape(B, ROW))
    return out.reshape(B, J, D)


# speedup vs baseline: 2.2983x; 1.5264x over previous
import functools

import jax
import jax.numpy as jnp
from jax import lax
from jax.experimental import pallas as pl
from jax.experimental.pallas import tpu as pltpu
from jax.experimental.pallas import tpu_sc as plsc

B, J, D = 16384, 26, 128
BB = 512

_mesh = plsc.VectorSubcoreMesh(core_axis_name="c", subcore_axis_name="s")


@functools.partial(
    pl.kernel,
    out_type=(jax.ShapeDtypeStruct((J, D), jnp.float32),
              jax.ShapeDtypeStruct((128,), jnp.float32)),
    mesh=_mesh,
    scratch_types=[
        pltpu.VMEM((128,), jnp.int32),
        pltpu.VMEM((J, 128), jnp.int32),
        pltpu.VMEM((128,), jnp.float32),
        pltpu.VMEM((J, 128), jnp.float32),
        pltpu.SemaphoreType.DMA,
    ],
)
def _sc_scales(idx_hbm, lw_hbm, rep_hbm, out_hbm, sig_hbm,
               idx_v, rep_v, w_v, sv_v, sem):
    cid = lax.axis_index("c")
    sid = lax.axis_index("s")

    @pl.when(jnp.logical_and(cid == 0, sid == 0))
    def _():
        pltpu.sync_copy(idx_hbm, idx_v)
        pltpu.sync_copy(rep_hbm, rep_v)
        pltpu.async_copy(lw_hbm.at[idx_v], w_v, sem).wait()
        for t in range(128 // 16):
            wv = w_v[pl.ds(16 * t, 16)]
            w_v[pl.ds(16 * t, 16)] = 1.0 / (1.0 + jnp.exp(-wv))
        pltpu.sync_copy(w_v, sig_hbm)
        descs = [
            pltpu.async_copy(sig_hbm.at[rep_v.at[q]], sv_v.at[q], sem)
            for q in range(J)
        ]
        for d in descs:
            d.wait()
        pltpu.sync_copy(sv_v, out_hbm)


def _tc_body(scale_ref, feat_ref, out_ref):
    out_ref[...] = feat_ref[...] * scale_ref[...]


_tc_mul = pl.pallas_call(
    _tc_body,
    grid=(B // BB,),
    in_specs=[
        pl.BlockSpec((1, J, D), lambda i: (0, 0, 0)),
        pl.BlockSpec((BB, J, D), lambda i: (i, 0, 0)),
    ],
    out_specs=pl.BlockSpec((BB, J, D), lambda i: (i, 0, 0)),
    out_shape=jax.ShapeDtypeStruct((B, J, D), jnp.float32),
)


def kernel(idx, feat, layerweight):
    idxp = jnp.zeros((128,), jnp.int32).at[:J].set(
        idx.reshape(J).astype(jnp.int32))
    lwp = jnp.zeros((128,), jnp.float32).at[:100].set(layerweight)
    rep = jnp.broadcast_to(jnp.arange(J, dtype=jnp.int32)[:, None], (J, 128))
    scale, _ = _sc_scales(idxp, lwp, rep)
    return _tc_mul(scale[None], feat)
